# baseline (speedup 1.0000x reference)
import functools

import jax
import jax.numpy as jnp
from jax.experimental import pallas as pl
from jax.experimental.pallas import tpu as pltpu

_CMID = 8
_K = 3
_NTAPS = _K * _K * _K


def _conv_pool_kernel(x_ref, w_ref, b_ref, psum_ref):
    dp2, hp2, wp2 = x_ref.shape[1], x_ref.shape[2], x_ref.shape[3]
    D, H, W = dp2 - 2, hp2 - 2, wp2 - 2

    def body(d, acc):
        slab = x_ref[0, pl.ds(d, 3)]
        taps = []
        for kd in range(_K):
            pk = slab[kd]
            for kh in range(_K):
                for kw in range(_K):
                    taps.append(pk[kh:kh + H, kw:kw + W])
        rows = []
        for c in range(_CMID):
            a = w_ref[c, 0] * taps[0]
            for t in range(1, _NTAPS):
                a = a + w_ref[c, t] * taps[t]
            a = jnp.maximum(a + b_ref[0, c], 0.0)
            rows.append(jnp.sum(a, axis=0, keepdims=True))
        return acc + jnp.concatenate(rows, axis=0)

    acc0 = jnp.zeros((_CMID, W), jnp.float32)
    psum_ref[0] = jax.lax.fori_loop(0, D, body, acc0)


def _head_kernel(inv_dhw, ps_ref, fcw_ref, fcb_ref, t_ref,
                 probs_ref, bce_ref, l1_ref, dice_ref):
    pooled = jnp.sum(ps_ref[...], axis=2) * inv_dhw
    z = jnp.sum(pooled * fcw_ref[...], axis=1, keepdims=True) + fcb_ref[0, 0]
    t = t_ref[...]

    e = jnp.exp(-jnp.abs(z))
    bce_ref[...] = jnp.mean(jnp.maximum(z, 0.0) - z * t + jnp.log1p(e),
                            keepdims=True)

    diff = z - t
    ad = jnp.abs(diff)
    l1_ref[...] = jnp.mean(jnp.where(ad < 1.0, 0.5 * diff * diff, ad - 0.5),
                           keepdims=True)

    inv1pe = 1.0 / (1.0 + e)
    p = jnp.where(z >= 0.0, inv1pe, e * inv1pe)

    inter = jnp.sum(p * t, keepdims=True)
    denom = jnp.sum(p, keepdims=True) + jnp.sum(t, keepdims=True)
    dice_ref[...] = 1.0 - (2.0 * inter + 1.0) / (denom + 1.0)

    probs_ref[...] = p


@jax.jit
def kernel(image, T_stage, conv_w, conv_b, fc_w, fc_b):
    B, _, D, H, W = image.shape
    xp = jnp.pad(image[:, 0].astype(jnp.float32),
                 ((0, 0), (1, 1), (1, 1), (1, 1)))
    w2 = conv_w.reshape(_CMID, _NTAPS).astype(jnp.float32)
    b2 = conv_b.reshape(1, _CMID).astype(jnp.float32)

    psum = pl.pallas_call(
        _conv_pool_kernel,
        out_shape=jax.ShapeDtypeStruct((B, _CMID, W), jnp.float32),
        grid=(B,),
        in_specs=[
            pl.BlockSpec((1, D + 2, H + 2, W + 2), lambda b: (b, 0, 0, 0)),
            pl.BlockSpec(memory_space=pltpu.MemorySpace.SMEM),
            pl.BlockSpec(memory_space=pltpu.MemorySpace.SMEM),
        ],
        out_specs=pl.BlockSpec((1, _CMID, W), lambda b: (b, 0, 0)),
        compiler_params=pltpu.CompilerParams(
            dimension_semantics=("parallel",)),
    )(xp, w2, b2)

    t_col = T_stage.astype(jnp.float32).reshape(B, 1)
    fcw_row = fc_w.reshape(1, _CMID).astype(jnp.float32)
    fcb = fc_b.reshape(1, 1).astype(jnp.float32)

    probs, bce, l1, dice = pl.pallas_call(
        functools.partial(_head_kernel, 1.0 / float(D * H * W)),
        out_shape=(
            jax.ShapeDtypeStruct((B, 1), jnp.float32),
            jax.ShapeDtypeStruct((1, 1), jnp.float32),
            jax.ShapeDtypeStruct((1, 1), jnp.float32),
            jax.ShapeDtypeStruct((1, 1), jnp.float32),
        ),
        in_specs=[
            pl.BlockSpec((B, _CMID, W), lambda: (0, 0, 0)),
            pl.BlockSpec((1, _CMID), lambda: (0, 0)),
            pl.BlockSpec((1, 1), lambda: (0, 0)),
            pl.BlockSpec((B, 1), lambda: (0, 0)),
        ],
        out_specs=(
            pl.BlockSpec((B, 1), lambda: (0, 0)),
            pl.BlockSpec((1, 1), lambda: (0, 0)),
            pl.BlockSpec((1, 1), lambda: (0, 0)),
            pl.BlockSpec((1, 1), lambda: (0, 0)),
        ),
    )(psum, fcw_row, fcb, t_col)

    return {
        'bce_loss': bce[0, 0],
        'l1s_loss': l1[0, 0],
        'dice_loss': dice[0, 0],
        'T_stage': probs.reshape(-1),
    }


# speedup vs baseline: 3.6386x; 3.6386x over previous
import functools

import jax
import jax.numpy as jnp
from jax.experimental import pallas as pl
from jax.experimental.pallas import tpu as pltpu

_CMID = 8
_K = 3
_NTAPS = _K * _K * _K


def _conv_pool_kernel(*refs):
    srcs, w_ref, b_ref, psum_ref = refs[:9], refs[9], refs[10], refs[11]
    D = srcs[0].shape[1] - 2
    W2 = srcs[0].shape[3]
    W = W2 // 2

    tap_idx = [(kd, kh, kw)
               for kd in range(_K) for kh in range(_K) for kw in range(_K)]
    wbf = [[jnp.bfloat16(w_ref[c, t]) for t in range(_NTAPS)]
           for c in range(_CMID)]
    bbf = [jnp.bfloat16(b_ref[0, c]) for c in range(_CMID)]

    def body(d, acc):
        accs = [None] * _CMID
        for t, (kd, kh, kw) in enumerate(tap_idx):
            tap = srcs[kh * _K + kw][0, d + kd]
            for c in range(_CMID):
                p = wbf[c][t] * tap
                accs[c] = p if t == 0 else accs[c] + p
        rows = []
        for c in range(_CMID):
            a = jnp.maximum(accs[c] + bbf[c], jnp.bfloat16(0.0))
            s = jnp.sum(a.astype(jnp.float32), axis=0, keepdims=True)
            rows.append(s[:, 0:W] + s[:, W:W2])
        return acc + jnp.concatenate(rows, axis=0)

    acc0 = jnp.zeros((_CMID, W), jnp.float32)
    psum_ref[0] = jax.lax.fori_loop(0, D, body, acc0)


def _head_kernel(inv_dhw, ps_ref, fcw_ref, fcb_ref, t_ref,
                 probs_ref, bce_ref, l1_ref, dice_ref):
    pooled = jnp.sum(ps_ref[...], axis=2) * inv_dhw
    z = jnp.sum(pooled * fcw_ref[...], axis=1, keepdims=True) + fcb_ref[0, 0]
    t = t_ref[...]

    e = jnp.exp(-jnp.abs(z))
    bce_ref[...] = jnp.mean(jnp.maximum(z, 0.0) - z * t + jnp.log1p(e),
                            keepdims=True)

    diff = z - t
    ad = jnp.abs(diff)
    l1_ref[...] = jnp.mean(jnp.where(ad < 1.0, 0.5 * diff * diff, ad - 0.5),
                           keepdims=True)

    inv1pe = 1.0 / (1.0 + e)
    p = jnp.where(z >= 0.0, inv1pe, e * inv1pe)

    inter = jnp.sum(p * t, keepdims=True)
    denom = jnp.sum(p, keepdims=True) + jnp.sum(t, keepdims=True)
    dice_ref[...] = 1.0 - (2.0 * inter + 1.0) / (denom + 1.0)

    probs_ref[...] = p


@jax.jit
def kernel(image, T_stage, conv_w, conv_b, fc_w, fc_b):
    B, _, D, H, W = image.shape
    Dp, Hp = D + 2, H + 2
    xp = jnp.pad(image[:, 0], ((0, 0), (1, 1), (1, 1), (1, 1))
                 ).astype(jnp.bfloat16)
    srcs = []
    for kh in range(_K):
        for kw in range(_K):
            sl = xp[:, :, kh:kh + H, kw:kw + W]
            srcs.append(sl.reshape(B, Dp, H // 2, 2 * W))
    w2 = (conv_w.reshape(_CMID, _NTAPS).astype(jnp.bfloat16)
          .astype(jnp.float32))
    b2 = conv_b.reshape(1, _CMID).astype(jnp.bfloat16).astype(jnp.float32)

    s_spec = pl.BlockSpec((1, Dp, H // 2, 2 * W), lambda b: (b, 0, 0, 0))
    psum = pl.pallas_call(
        _conv_pool_kernel,
        out_shape=jax.ShapeDtypeStruct((B, _CMID, W), jnp.float32),
        grid=(B,),
        in_specs=[s_spec] * 9 + [
            pl.BlockSpec(memory_space=pltpu.MemorySpace.SMEM),
            pl.BlockSpec(memory_space=pltpu.MemorySpace.SMEM),
        ],
        out_specs=pl.BlockSpec((1, _CMID, W), lambda b: (b, 0, 0)),
        compiler_params=pltpu.CompilerParams(
            dimension_semantics=("parallel",)),
    )(*srcs, w2, b2)

    t_col = T_stage.astype(jnp.float32).reshape(B, 1)
    fcw_row = fc_w.reshape(1, _CMID).astype(jnp.float32)
    fcb = fc_b.reshape(1, 1).astype(jnp.float32)

    probs, bce, l1, dice = pl.pallas_call(
        functools.partial(_head_kernel, 1.0 / float(D * H * W)),
        out_shape=(
            jax.ShapeDtypeStruct((B, 1), jnp.float32),
            jax.ShapeDtypeStruct((1, 1), jnp.float32),
            jax.ShapeDtypeStruct((1, 1), jnp.float32),
            jax.ShapeDtypeStruct((1, 1), jnp.float32),
        ),
        in_specs=[
            pl.BlockSpec((B, _CMID, W), lambda: (0, 0, 0)),
            pl.BlockSpec((1, _CMID), lambda: (0, 0)),
            pl.BlockSpec((1, 1), lambda: (0, 0)),
            pl.BlockSpec((B, 1), lambda: (0, 0)),
        ],
        out_specs=(
            pl.BlockSpec((B, 1), lambda: (0, 0)),
            pl.BlockSpec((1, 1), lambda: (0, 0)),
            pl.BlockSpec((1, 1), lambda: (0, 0)),
            pl.BlockSpec((1, 1), lambda: (0, 0)),
        ),
    )(psum, fcw_row, fcb, t_col)

    return {
        'bce_loss': bce[0, 0],
        'l1s_loss': l1[0, 0],
        'dice_loss': dice[0, 0],
        'T_stage': probs.reshape(-1),
    }
